# baseline (speedup 1.0000x reference)
import functools

import jax
import jax.numpy as jnp
from jax import lax
from jax.experimental import pallas as pl
from jax.experimental.pallas import tpu as pltpu
from jax.experimental.pallas import tpu_sc as plsc

V_TILE = 2048


def _gather(table, idx):
    n, per_tok = idx.shape[0], table.shape[1]
    info = plsc.get_sparse_core_info()
    nw = info.num_cores * info.num_subcores
    b_per_w = n // nw
    mesh = plsc.VectorSubcoreMesh(core_axis_name="c", subcore_axis_name="s")

    chunk = 16

    @functools.partial(
        pl.kernel,
        out_type=jax.ShapeDtypeStruct((n, per_tok), jnp.float32),
        mesh=mesh,
        scratch_types=[
            pltpu.VMEM((b_per_w,), jnp.int32),
            pltpu.VMEM((b_per_w, per_tok), jnp.float32),
            pltpu.SemaphoreType.DMA,
        ],
    )
    def gather_kernel(idx_hbm, table_hbm, out_hbm, idx_s, rows_v, sem):
        wid = lax.axis_index("s") * info.num_cores + lax.axis_index("c")
        base = wid * b_per_w
        pltpu.sync_copy(idx_hbm.at[pl.ds(base, b_per_w)], idx_s)

        def body(ci, _):
            base_i = ci * chunk
            v = idx_s[pl.ds(base_i, chunk)]
            copies = []
            for j in range(chunk):
                copies.append(
                    pltpu.async_copy(
                        table_hbm.at[v[j]], rows_v.at[base_i + j], sem))
            for c in copies:
                c.wait()
            return ()

        lax.fori_loop(0, b_per_w // chunk, body, (), unroll=False)
        pltpu.sync_copy(rows_v, out_hbm.at[pl.ds(base, b_per_w)])

    return gather_kernel(idx, table)


def _h1_body(emb_ref, w1_ref, b1_ref, h1_ref):
    h1 = lax.dot_general(
        emb_ref[...], w1_ref[...], (((1,), (1,)), ((), ())),
        preferred_element_type=jnp.float32)
    h1_ref[...] = jnp.maximum(h1 + b1_ref[...], 0.0).astype(jnp.bfloat16)


def _h1(emb, W1, b1):
    batch = emb.shape[0]
    hid = W1.shape[0]
    return pl.pallas_call(
        _h1_body,
        out_shape=jax.ShapeDtypeStruct((batch, hid), jnp.bfloat16),
    )(emb, W1, b1.reshape(1, hid))


NQ = 4
NSLOT = 3
NR = 4


def _mm2_body(vocab, batch, hid, n_static, h1_hbm, w2_hbm, b2_hbm, out_ref,
              h1buf, b2buf, w2buf, obuf, tbuf, sems, tsems, rsems, lsem):
    i = pl.program_id(0)
    n = pl.num_programs(0)
    slot = lax.rem(i, NSLOT)
    ri = lax.rem(i, NR)
    rows = batch // NQ
    tail = vocab - (n - 1) * V_TILE

    def w2_load(t, s, height):
        return pltpu.make_async_copy(
            w2_hbm.at[pl.ds(t * V_TILE, height)],
            w2buf.at[s, pl.ds(0, height)],
            rsems.at[s])

    @pl.when(i == 0)
    def _():
        pltpu.async_copy(h1_hbm, h1buf, lsem).wait()
        pltpu.async_copy(b2_hbm, b2buf, lsem).wait()
        for r in range(NR):
            w2_load(r, r, V_TILE).start()

    @pl.when(i < n - 1)
    def _():
        w2_load(i, ri, V_TILE).wait()

    @pl.when(i == n - 1)
    def _():
        w2_load(i, ri, tail).wait()

    def waitq(s, width):
        for q in range(NQ):
            pltpu.make_async_copy(
                obuf.at[s, pl.ds(q * rows, rows), pl.ds(0, width)],
                out_ref.at[pl.ds(q * rows, rows), pl.ds(0, width)],
                sems.at[s, q]).wait()

    @pl.when(i >= NSLOT)
    def _():
        waitq(slot, V_TILE)

    w2b = w2buf[ri].astype(jnp.bfloat16)
    res = lax.dot_general(
        h1buf[...], w2b, (((1,), (1,)), ((), ())),
        preferred_element_type=jnp.float32) + b2buf[lax.rem(i, n), :]

    @pl.when(i < n - 1)
    def _():
        obuf[slot] = res
        off = i * V_TILE
        for q in range(NQ):
            pltpu.make_async_copy(
                obuf.at[slot, pl.ds(q * rows, rows)],
                out_ref.at[pl.ds(q * rows, rows), pl.ds(off, V_TILE)],
                sems.at[slot, q]).start()

    @pl.when(jnp.logical_and(i + NR < n - 1, i + NR >= NR))
    def _():
        w2_load(i + NR, ri, V_TILE).start()

    @pl.when(i + NR == n - 1)
    def _():
        w2_load(i + NR, ri, tail).start()

    @pl.when(i == n - 1)
    def _():
        tbuf[...] = res[:, :tail]
        off = (n - 1) * V_TILE
        for q in range(NQ):
            pltpu.make_async_copy(
                tbuf.at[pl.ds(q * rows, rows)],
                out_ref.at[pl.ds(q * rows, rows), pl.ds(off, tail)],
                tsems.at[q]).start()
        last_slot = (n_static - 1) % NSLOT
        for s in range(NSLOT):
            if s != last_slot:
                waitq(s, V_TILE)
        for q in range(NQ):
            pltpu.make_async_copy(
                tbuf.at[pl.ds(q * rows, rows)],
                out_ref.at[pl.ds(q * rows, rows), pl.ds(off, tail)],
                tsems.at[q]).wait()


def _mm2(h1b, W2, b2):
    batch, hid = h1b.shape
    vocab = W2.shape[0]
    n = pl.cdiv(vocab, V_TILE)
    tail = vocab - (n - 1) * V_TILE
    b2p = jnp.zeros((n, V_TILE), jnp.float32).at[
        :, :].set(0.0).reshape(-1).at[:vocab].set(b2).reshape(n, V_TILE)
    body = functools.partial(_mm2_body, vocab, batch, hid, n)
    return pl.pallas_call(
        body,
        grid=(n,),
        in_specs=[
            pl.BlockSpec(memory_space=pl.ANY),
            pl.BlockSpec(memory_space=pl.ANY),
            pl.BlockSpec(memory_space=pl.ANY),
        ],
        out_specs=pl.BlockSpec(memory_space=pl.ANY),
        out_shape=jax.ShapeDtypeStruct((batch, vocab), jnp.float32),
        scratch_shapes=[
            pltpu.VMEM((batch, hid), jnp.bfloat16),
            pltpu.VMEM((n, V_TILE), jnp.float32),
            pltpu.VMEM((NR, V_TILE, hid), jnp.float32),
            pltpu.VMEM((NSLOT, batch, V_TILE), jnp.float32),
            pltpu.VMEM((batch, tail), jnp.float32),
            pltpu.SemaphoreType.DMA((NSLOT, NQ)),
            pltpu.SemaphoreType.DMA((NQ,)),
            pltpu.SemaphoreType.DMA((NR,)),
            pltpu.SemaphoreType.DMA,
        ],
    )(h1b, W2, b2p)


def kernel(inputs, table, W1, b1, W2, b2):
    batch, ctx = inputs.shape
    idx = inputs.reshape(-1).astype(jnp.int32)
    emb = _gather(table, idx).reshape(batch, ctx * table.shape[1])
    h1b = _h1(emb, W1, b1)
    return _mm2(h1b, W2, b2)


# speedup vs baseline: 1.0014x; 1.0014x over previous
import functools

import jax
import jax.numpy as jnp
from jax import lax
from jax.experimental import pallas as pl
from jax.experimental.pallas import tpu as pltpu
from jax.experimental.pallas import tpu_sc as plsc

V_TILE = 2048


def _gather(table, idx):
    n, per_tok = idx.shape[0], table.shape[1]
    info = plsc.get_sparse_core_info()
    nw = info.num_cores * info.num_subcores
    b_per_w = n // nw
    mesh = plsc.VectorSubcoreMesh(core_axis_name="c", subcore_axis_name="s")

    chunk = 16

    @functools.partial(
        pl.kernel,
        out_type=jax.ShapeDtypeStruct((n, per_tok), jnp.float32),
        mesh=mesh,
        scratch_types=[
            pltpu.VMEM((b_per_w,), jnp.int32),
            pltpu.VMEM((b_per_w, per_tok), jnp.float32),
            pltpu.SemaphoreType.DMA,
        ],
    )
    def gather_kernel(idx_hbm, table_hbm, out_hbm, idx_s, rows_v, sem):
        wid = lax.axis_index("s") * info.num_cores + lax.axis_index("c")
        base = wid * b_per_w
        pltpu.sync_copy(idx_hbm.at[pl.ds(base, b_per_w)], idx_s)

        def body(ci, _):
            base_i = ci * chunk
            v = idx_s[pl.ds(base_i, chunk)]
            copies = []
            for j in range(chunk):
                copies.append(
                    pltpu.async_copy(
                        table_hbm.at[v[j]], rows_v.at[base_i + j], sem))
            for c in copies:
                c.wait()
            return ()

        lax.fori_loop(0, b_per_w // chunk, body, (), unroll=False)
        pltpu.sync_copy(rows_v, out_hbm.at[pl.ds(base, b_per_w)])

    return gather_kernel(idx, table)


def _h1_body(emb_ref, w1_ref, b1_ref, h1_ref):
    h1 = lax.dot_general(
        emb_ref[...], w1_ref[...], (((1,), (1,)), ((), ())),
        preferred_element_type=jnp.float32)
    h1_ref[...] = jnp.maximum(h1 + b1_ref[...], 0.0).astype(jnp.bfloat16)


def _h1(emb, W1, b1):
    batch = emb.shape[0]
    hid = W1.shape[0]
    return pl.pallas_call(
        _h1_body,
        out_shape=jax.ShapeDtypeStruct((batch, hid), jnp.bfloat16),
    )(emb, W1, b1.reshape(1, hid))


NQ = 4
NSLOT = 3


def _mm2_body(vocab, batch, hid, n, h1_hbm, w2_hbm, b2_hbm, out_ref,
              h1buf, b2buf, w2buf, obuf, tbuf, sems, tsems, rsems, lsem):
    rows = batch // NQ
    tail = vocab - (n - 1) * V_TILE
    n_full = n - 1
    n_super = n_full // NSLOT

    def w2_load(t, s, height):
        return pltpu.make_async_copy(
            w2_hbm.at[pl.ds(t * V_TILE, height)],
            w2buf.at[s, pl.ds(0, height)],
            rsems.at[s])

    def store(s, t, q):
        return pltpu.make_async_copy(
            obuf.at[s, pl.ds(q * rows, rows)],
            out_ref.at[pl.ds(q * rows, rows), pl.ds(t * V_TILE, V_TILE)],
            sems.at[s, q])

    pltpu.async_copy(h1_hbm, h1buf, lsem).wait()
    pltpu.async_copy(b2_hbm, b2buf, lsem).wait()
    for s in range(NSLOT):
        w2_load(s, s, V_TILE).start()

    def superstep(u, _):
        for s in range(NSLOT):
            t = u * NSLOT + s
            w2_load(t, s, V_TILE).wait()
            res = lax.dot_general(
                h1buf[...], w2buf[s].astype(jnp.bfloat16),
                (((1,), (1,)), ((), ())),
                preferred_element_type=jnp.float32) + b2buf[t, :]

            @pl.when(u >= 1)
            def _():
                for q in range(NQ):
                    store(s, 0, q).wait()

            obuf[s] = res
            for q in range(NQ):
                store(s, t, q).start()

            nxt = t + NSLOT

            @pl.when(nxt < n_full)
            def _():
                w2_load(nxt, s, V_TILE).start()

            @pl.when(nxt == n_full)
            def _():
                w2_load(nxt, s, tail).start()
        return ()

    lax.fori_loop(0, n_super, superstep, (), unroll=False)

    for t in range(n_super * NSLOT, n_full):
        s = t % NSLOT
        w2_load(t, s, V_TILE).wait()
        res = lax.dot_general(
            h1buf[...], w2buf[s].astype(jnp.bfloat16),
            (((1,), (1,)), ((), ())),
            preferred_element_type=jnp.float32) + b2buf[t, :]
        for q in range(NQ):
            store(s, 0, q).wait()
        obuf[s] = res
        for q in range(NQ):
            store(s, t, q).start()

    t = n_full
    s = t % NSLOT
    w2_load(t, s, tail).wait()
    res = lax.dot_general(
        h1buf[...], w2buf[s].astype(jnp.bfloat16),
        (((1,), (1,)), ((), ())),
        preferred_element_type=jnp.float32) + b2buf[t, :]
    tbuf[...] = res[:, :tail]
    for q in range(NQ):
        pltpu.make_async_copy(
            tbuf.at[pl.ds(q * rows, rows)],
            out_ref.at[pl.ds(q * rows, rows), pl.ds(t * V_TILE, tail)],
            tsems.at[q]).start()
    for s2 in range(NSLOT):
        for q in range(NQ):
            store(s2, 0, q).wait()
    for q in range(NQ):
        pltpu.make_async_copy(
            tbuf.at[pl.ds(q * rows, rows)],
            out_ref.at[pl.ds(q * rows, rows), pl.ds(t * V_TILE, tail)],
            tsems.at[q]).wait()


def _mm2(h1b, W2, b2):
    batch, hid = h1b.shape
    vocab = W2.shape[0]
    n = pl.cdiv(vocab, V_TILE)
    tail = vocab - (n - 1) * V_TILE
    b2p = jnp.zeros((n * V_TILE,), jnp.float32).at[:vocab].set(
        b2).reshape(n, V_TILE)
    body = functools.partial(_mm2_body, vocab, batch, hid, n)
    return pl.pallas_call(
        body,
        in_specs=[
            pl.BlockSpec(memory_space=pl.ANY),
            pl.BlockSpec(memory_space=pl.ANY),
            pl.BlockSpec(memory_space=pl.ANY),
        ],
        out_specs=pl.BlockSpec(memory_space=pl.ANY),
        out_shape=jax.ShapeDtypeStruct((batch, vocab), jnp.float32),
        scratch_shapes=[
            pltpu.VMEM((batch, hid), jnp.bfloat16),
            pltpu.VMEM((n, V_TILE), jnp.float32),
            pltpu.VMEM((NSLOT, V_TILE, hid), jnp.float32),
            pltpu.VMEM((NSLOT, batch, V_TILE), jnp.float32),
            pltpu.VMEM((batch, tail), jnp.float32),
            pltpu.SemaphoreType.DMA((NSLOT, NQ)),
            pltpu.SemaphoreType.DMA((NQ,)),
            pltpu.SemaphoreType.DMA((NSLOT,)),
            pltpu.SemaphoreType.DMA,
        ],
    )(h1b, W2, b2p)


def kernel(inputs, table, W1, b1, W2, b2):
    batch, ctx = inputs.shape
    idx = inputs.reshape(-1).astype(jnp.int32)
    emb = _gather(table, idx).reshape(batch, ctx * table.shape[1])
    h1b = _h1(emb, W1, b1)
    return _mm2(h1b, W2, b2)
